# baseline (speedup 1.0000x reference)
import functools

import jax
import jax.numpy as jnp
from jax import lax
from jax.experimental import pallas as pl
from jax.experimental.pallas import tpu as pltpu
from jax.experimental.pallas import tpu_sc as plsc

IN_DIM = 2048
LATENT_DIM = 256
NUM_TOKENS = 8192
TOK_TILE = 1024
K_TILE = 1024


def _argmax_body(x_ref, wp_ref, b_ref, wc_ref, idx_ref):
    z = lax.dot_general(
        x_ref[...], wp_ref[...],
        (((1,), (1,)), ((), ())),
        preferred_element_type=jnp.float32,
    ) + b_ref[...]
    acc = jnp.full((TOK_TILE, K_TILE), -jnp.inf, dtype=jnp.float32)
    acc_k = jnp.zeros((TOK_TILE, K_TILE), dtype=jnp.int32)
    for kt in range(NUM_TOKENS // K_TILE):
        logits_t = lax.dot_general(
            z, wc_ref[pl.ds(kt * K_TILE, K_TILE), :],
            (((1,), (1,)), ((), ())),
            preferred_element_type=jnp.float32,
        )
        gt = logits_t > acc
        acc = jnp.maximum(acc, logits_t)
        acc_k = jnp.where(gt, kt, acc_k)
    m = jnp.max(acc, axis=-1, keepdims=True)
    jj = lax.broadcasted_iota(jnp.int32, (TOK_TILE, K_TILE), 1)
    cand = jnp.where(acc == m, acc_k * K_TILE + jj, NUM_TOKENS)
    idx_ref[...] = jnp.min(cand, axis=-1)[None, None, :]


def _compute_indices(xf, W_proj, b_proj, W_code):
    n_tokens = xf.shape[0]
    grid = n_tokens // TOK_TILE
    idx3 = pl.pallas_call(
        _argmax_body,
        grid=(grid,),
        in_specs=[
            pl.BlockSpec((TOK_TILE, IN_DIM), lambda i: (i, 0)),
            pl.BlockSpec((LATENT_DIM, IN_DIM), lambda i: (0, 0)),
            pl.BlockSpec((1, LATENT_DIM), lambda i: (0, 0)),
            pl.BlockSpec((NUM_TOKENS, LATENT_DIM), lambda i: (0, 0)),
        ],
        out_specs=pl.BlockSpec((1, 1, TOK_TILE), lambda i: (i, 0, 0)),
        out_shape=jax.ShapeDtypeStruct((grid, 1, TOK_TILE), jnp.int32),
    )(xf, W_proj, b_proj.reshape(1, LATENT_DIM), W_code)
    return idx3.reshape(n_tokens)


_NC, _NS = 2, 16
_NW = _NC * _NS


def _make_sc_gather(n_tokens):
    b_per_w = n_tokens // _NW
    mesh = plsc.VectorSubcoreMesh(core_axis_name="c", subcore_axis_name="s")

    @functools.partial(
        pl.kernel, mesh=mesh,
        out_type=jax.ShapeDtypeStruct((n_tokens, LATENT_DIM), jnp.float32),
        scratch_types=[
            pltpu.VMEM((b_per_w,), jnp.int32),
            pltpu.VMEM((b_per_w, LATENT_DIM), jnp.float32),
            pltpu.SemaphoreType.DMA,
        ],
    )
    def gather(table_hbm, idx_hbm, out_hbm, idx_v, rows_v, sem):
        wid = lax.axis_index("s") * _NC + lax.axis_index("c")
        base = wid * b_per_w
        pltpu.sync_copy(idx_hbm.at[pl.ds(base, b_per_w)], idx_v)
        pltpu.async_copy(table_hbm.at[idx_v], rows_v, sem).wait()
        pltpu.sync_copy(rows_v, out_hbm.at[pl.ds(base, b_per_w)])

    return gather


def kernel(x, W_proj, b_proj, W_code):
    B_, S_, _ = x.shape
    n_tokens = B_ * S_
    xf = x.reshape(n_tokens, IN_DIM)
    idx = _compute_indices(xf, W_proj, b_proj, W_code)
    codes = _make_sc_gather(n_tokens)(W_code, idx)
    return (codes.reshape(B_, S_, LATENT_DIM), idx.reshape(B_, S_, 1))


# speedup vs baseline: 1.2109x; 1.2109x over previous
import functools

import jax
import jax.numpy as jnp
from jax import lax
from jax.experimental import pallas as pl
from jax.experimental.pallas import tpu as pltpu
from jax.experimental.pallas import tpu_sc as plsc

IN_DIM = 2048
LATENT_DIM = 256
NUM_TOKENS = 8192
TOK_TILE = 1024


def _argmax_body(x_ref, wp_ref, b_ref, wc_ref, idx_ref):
    z = lax.dot_general(
        x_ref[...], wp_ref[...],
        (((1,), (1,)), ((), ())),
        preferred_element_type=jnp.float32,
    ) + b_ref[...]
    logits = lax.dot_general(
        z, wc_ref[...],
        (((1,), (1,)), ((), ())),
        preferred_element_type=jnp.float32,
    )
    idx_ref[...] = jnp.argmax(logits, axis=-1).astype(jnp.int32)[None, None, :]


def _compute_indices(xf, W_proj, b_proj, W_code):
    n_tokens = xf.shape[0]
    grid = n_tokens // TOK_TILE
    idx3 = pl.pallas_call(
        _argmax_body,
        grid=(grid,),
        in_specs=[
            pl.BlockSpec((TOK_TILE, IN_DIM), lambda i: (i, 0)),
            pl.BlockSpec((LATENT_DIM, IN_DIM), lambda i: (0, 0)),
            pl.BlockSpec((1, LATENT_DIM), lambda i: (0, 0)),
            pl.BlockSpec((NUM_TOKENS, LATENT_DIM), lambda i: (0, 0)),
        ],
        out_specs=pl.BlockSpec((1, 1, TOK_TILE), lambda i: (i, 0, 0)),
        out_shape=jax.ShapeDtypeStruct((grid, 1, TOK_TILE), jnp.int32),
        compiler_params=pltpu.CompilerParams(
            vmem_limit_bytes=100 * 1024 * 1024,
        ),
    )(xf, W_proj, b_proj.reshape(1, LATENT_DIM), W_code)
    return idx3.reshape(n_tokens)


_NC, _NS = 2, 16
_NW = _NC * _NS


def _make_sc_gather(n_tokens):
    b_per_w = n_tokens // _NW
    mesh = plsc.VectorSubcoreMesh(core_axis_name="c", subcore_axis_name="s")

    half = b_per_w // 2

    @functools.partial(
        pl.kernel, mesh=mesh,
        out_type=jax.ShapeDtypeStruct((n_tokens, LATENT_DIM), jnp.float32),
        scratch_types=[
            pltpu.VMEM((2, half), jnp.int32),
            pltpu.VMEM((2, half, LATENT_DIM), jnp.float32),
            pltpu.SemaphoreType.DMA,
            pltpu.SemaphoreType.DMA,
        ],
    )
    def gather(table_hbm, idx_hbm, out_hbm, idx_v, rows_v, sem0, sem1):
        wid = lax.axis_index("s") * _NC + lax.axis_index("c")
        base = wid * b_per_w
        pltpu.sync_copy(idx_hbm.at[pl.ds(base, half)], idx_v.at[0])
        cp0 = pltpu.async_copy(table_hbm.at[idx_v.at[0]], rows_v.at[0], sem0)
        pltpu.sync_copy(idx_hbm.at[pl.ds(base + half, half)], idx_v.at[1])
        cp1 = pltpu.async_copy(table_hbm.at[idx_v.at[1]], rows_v.at[1], sem1)
        cp0.wait()
        pltpu.sync_copy(rows_v.at[0], out_hbm.at[pl.ds(base, half)])
        cp1.wait()
        pltpu.sync_copy(rows_v.at[1], out_hbm.at[pl.ds(base + half, half)])

    return gather


def kernel(x, W_proj, b_proj, W_code):
    B_, S_, _ = x.shape
    n_tokens = B_ * S_
    xf = x.reshape(n_tokens, IN_DIM)
    idx = _compute_indices(xf, W_proj, b_proj, W_code)
    codes = _make_sc_gather(n_tokens)(W_code, idx)
    return (codes.reshape(B_, S_, LATENT_DIM), idx.reshape(B_, S_, 1))


# speedup vs baseline: 1.2135x; 1.0022x over previous
import functools

import jax
import jax.numpy as jnp
from jax import lax
from jax.experimental import pallas as pl
from jax.experimental.pallas import tpu as pltpu
from jax.experimental.pallas import tpu_sc as plsc

IN_DIM = 2048
LATENT_DIM = 256
NUM_TOKENS = 8192
TOK_TILE = 1024


def _argmax_body(x_ref, wp_ref, b_ref, wc_ref, idx_ref):
    z = lax.dot_general(
        x_ref[...], wp_ref[...],
        (((1,), (1,)), ((), ())),
        preferred_element_type=jnp.float32,
    ) + b_ref[...]
    logits = lax.dot_general(
        z, wc_ref[...],
        (((1,), (1,)), ((), ())),
        preferred_element_type=jnp.float32,
    )
    idx_ref[...] = jnp.argmax(logits, axis=-1).astype(jnp.int32)[None, None, :]


def _compute_indices(xf, W_proj, b_proj, W_code):
    n_tokens = xf.shape[0]
    grid = n_tokens // TOK_TILE
    idx3 = pl.pallas_call(
        _argmax_body,
        grid=(grid,),
        in_specs=[
            pl.BlockSpec((TOK_TILE, IN_DIM), lambda i: (i, 0)),
            pl.BlockSpec((LATENT_DIM, IN_DIM), lambda i: (0, 0)),
            pl.BlockSpec((1, LATENT_DIM), lambda i: (0, 0)),
            pl.BlockSpec((NUM_TOKENS, LATENT_DIM), lambda i: (0, 0)),
        ],
        out_specs=pl.BlockSpec((1, 1, TOK_TILE), lambda i: (i, 0, 0)),
        out_shape=jax.ShapeDtypeStruct((grid, 1, TOK_TILE), jnp.int32),
    )(xf, W_proj, b_proj.reshape(1, LATENT_DIM), W_code)
    return idx3.reshape(n_tokens)


_NC, _NS = 2, 16
_NW = _NC * _NS


def _make_sc_gather(n_tokens):
    b_per_w = n_tokens // _NW
    mesh = plsc.VectorSubcoreMesh(core_axis_name="c", subcore_axis_name="s")

    half = b_per_w // 2

    @functools.partial(
        pl.kernel, mesh=mesh,
        out_type=jax.ShapeDtypeStruct((n_tokens, LATENT_DIM), jnp.float32),
        scratch_types=[
            pltpu.VMEM((2, half), jnp.int32),
            pltpu.VMEM((2, half, LATENT_DIM), jnp.float32),
            pltpu.SemaphoreType.DMA,
            pltpu.SemaphoreType.DMA,
        ],
    )
    def gather(table_hbm, idx_hbm, out_hbm, idx_v, rows_v, sem0, sem1):
        wid = lax.axis_index("s") * _NC + lax.axis_index("c")
        base = wid * b_per_w
        pltpu.sync_copy(idx_hbm.at[pl.ds(base, half)], idx_v.at[0])
        cp0 = pltpu.async_copy(table_hbm.at[idx_v.at[0]], rows_v.at[0], sem0)
        pltpu.sync_copy(idx_hbm.at[pl.ds(base + half, half)], idx_v.at[1])
        cp1 = pltpu.async_copy(table_hbm.at[idx_v.at[1]], rows_v.at[1], sem1)
        cp0.wait()
        pltpu.sync_copy(rows_v.at[0], out_hbm.at[pl.ds(base, half)])
        cp1.wait()
        pltpu.sync_copy(rows_v.at[1], out_hbm.at[pl.ds(base + half, half)])

    return gather


def kernel(x, W_proj, b_proj, W_code):
    B_, S_, _ = x.shape
    n_tokens = B_ * S_
    xf = x.reshape(n_tokens, IN_DIM)
    idx = _compute_indices(xf, W_proj, b_proj, W_code)
    codes = _make_sc_gather(n_tokens)(W_code, idx)
    return (codes.reshape(B_, S_, LATENT_DIM), idx.reshape(B_, S_, 1))
